# Initial kernel scaffold; baseline (speedup 1.0000x reference)
import jax
import jax.numpy as jnp
from jax.experimental import pallas as pl


def kernel(time_seqs, time_delta_seqs, type_seqs, num_sample):
    raise NotImplementedError("write your pallas kernel here")


# speedup vs baseline: 2.7631x; 2.7631x over previous
import functools

import jax
import jax.numpy as jnp
from jax.experimental import pallas as pl

_NUM_TYPES = 10
_E = 32
_K = 16
_OVER = 5.0
_TL = 512

_MU = (0.10000000149011612, 0.20000000298023224, 0.30000001192092896,
       0.4000000059604645, 0.5, 0.6000000238418579, 0.699999988079071,
       0.800000011920929, 0.8999999761581421, 1.0)


def _rotl(x, r):
    return (x << jnp.uint32(r)) | (x >> jnp.uint32(32 - r))


def _threefry_bits(k1_int, x1):
    k1 = jnp.uint32(k1_int)
    ks2 = jnp.uint32(0x1BD11BDA) ^ k1
    zero = jnp.uint32(0)
    x0 = jnp.zeros_like(x1)
    x1 = x1 + k1
    rots0 = (13, 15, 26, 6)
    rots1 = (17, 29, 16, 24)
    inj = ((k1, ks2), (ks2, zero), (zero, k1), (k1, ks2), (ks2, zero))
    for g in range(5):
        for r in (rots0 if g % 2 == 0 else rots1):
            x0 = x0 + x1
            x1 = _rotl(x1, r)
            x1 = x1 ^ x0
        a, bb = inj[g]
        x0 = x0 + a
        x1 = x1 + (bb + jnp.uint32(g + 1))
    return x0 ^ x1


def _bits_to_uniform(bits):
    f = jax.lax.bitcast_convert_type(
        (bits >> jnp.uint32(9)) | jnp.uint32(0x3F800000), jnp.float32)
    return f - jnp.float32(1.0)


def _body(t_ref, dt_ref, ty_ref, out_ref, *, L):
    b = pl.program_id(0)
    lt = pl.program_id(1)
    t = t_ref[0]
    dt = dt_ref[0]
    ty = ty_ref[0]

    te = jnp.zeros_like(t)
    for k in range(_NUM_TYPES):
        te = te + jnp.where(ty == k, jnp.float32(_MU[k]), jnp.float32(0.0))

    base = jnp.float32(0.1) + jax.nn.softplus(
        te + jnp.float32(0.1) * dt + jnp.float32(0.01) * jnp.cos(t))

    v0 = jnp.zeros_like(base)
    for k in range(_NUM_TYPES):
        v0 = v0 + (base * jnp.float32(_MU[k]) + jnp.float32(0.05))
    M = v0 * jnp.float32(_OVER)

    sub = jax.lax.broadcasted_iota(jnp.int32, (_E, _TL), 0)
    lane = jax.lax.broadcasted_iota(jnp.int32, (_E, _TL), 1)
    l0 = lt * _TL
    ie = (b * (L * _E) + (l0 + lane) * _E + sub).astype(jnp.uint32)
    u1 = _bits_to_uniform(_threefry_bits(1, ie))
    e = -jnp.log1p(-u1)
    x = e / M
    for s in (1, 2, 4, 8, 16):
        shifted = jnp.concatenate(
            [jnp.zeros((s, _TL), jnp.float32), x[:-s, :]], axis=0)
        x = x + shifted
    exp_j = x

    st = base * jnp.exp(jnp.float32(-0.5) * exp_j)
    intens = jnp.zeros_like(st)
    for k in range(_NUM_TYPES):
        intens = intens + (st * jnp.float32(_MU[k]) + jnp.float32(0.05))

    rows = []
    big = jnp.float32(jnp.inf)
    for k in range(_K):
        iu = (b * (L * _K * _E) + (l0 + lane) * (_K * _E)
              + k * _E + sub).astype(jnp.uint32)
        u = _bits_to_uniform(_threefry_bits(2, iu))
        crit = (u * M) / intens
        cand = jnp.where(crit < jnp.float32(1.0), exp_j, big)
        mval = jnp.min(cand, axis=0, keepdims=True)
        res = jnp.where(mval == big, jnp.float32(0.0),
                        jnp.minimum(mval, jnp.float32(100000.0)))
        rows.append(res)
    out_ref[...] = jnp.concatenate(rows, axis=0)


def kernel(time_seqs, time_delta_seqs, type_seqs, num_sample):
    B, L = time_seqs.shape
    in_spec = pl.BlockSpec((1, 1, _TL), lambda b, lt: (b, 0, lt))
    out = pl.pallas_call(
        functools.partial(_body, L=L),
        grid=(B, L // _TL),
        in_specs=[in_spec, in_spec, in_spec],
        out_specs=pl.BlockSpec((_K, _TL), lambda b, lt: (b, lt)),
        out_shape=jax.ShapeDtypeStruct((B * _K, L), jnp.float32),
    )(time_seqs.reshape(B, 1, L), time_delta_seqs.reshape(B, 1, L),
      type_seqs.reshape(B, 1, L))
    res = out.reshape(B, _K, L).transpose(0, 2, 1)
    weights = jnp.ones((B, L, _K), jnp.float32) / num_sample
    return (res, weights)


# speedup vs baseline: 2.7632x; 1.0000x over previous
import functools

import jax
import jax.numpy as jnp
from jax.experimental import pallas as pl
from jax.experimental.pallas import tpu as pltpu

_NUM_TYPES = 10
_E = 32
_K = 16
_OVER = 5.0
_TL = 512

_MU = (0.10000000149011612, 0.20000000298023224, 0.30000001192092896,
       0.4000000059604645, 0.5, 0.6000000238418579, 0.699999988079071,
       0.800000011920929, 0.8999999761581421, 1.0)


def _rotl(x, r):
    return (x << jnp.uint32(r)) | (x >> jnp.uint32(32 - r))


def _threefry_bits(k1_int, x1):
    k1 = jnp.uint32(k1_int)
    ks2 = jnp.uint32(0x1BD11BDA) ^ k1
    zero = jnp.uint32(0)
    x0 = jnp.zeros_like(x1)
    x1 = x1 + k1
    rots0 = (13, 15, 26, 6)
    rots1 = (17, 29, 16, 24)
    inj = ((k1, ks2), (ks2, zero), (zero, k1), (k1, ks2), (ks2, zero))
    for g in range(5):
        for r in (rots0 if g % 2 == 0 else rots1):
            x0 = x0 + x1
            x1 = _rotl(x1, r)
            x1 = x1 ^ x0
        a, bb = inj[g]
        x0 = x0 + a
        x1 = x1 + (bb + jnp.uint32(g + 1))
    return x0 ^ x1


def _bits_to_uniform(bits):
    f = jax.lax.bitcast_convert_type(
        (bits >> jnp.uint32(9)) | jnp.uint32(0x3F800000), jnp.float32)
    return f - jnp.float32(1.0)


def _body(t_ref, dt_ref, ty_ref, out_ref, *, L):
    b = pl.program_id(0)
    lt = pl.program_id(1)
    t = t_ref[0]
    dt = dt_ref[0]
    ty = ty_ref[0]

    te = jnp.zeros_like(t)
    for k in range(_NUM_TYPES):
        te = te + jnp.where(ty == k, jnp.float32(_MU[k]), jnp.float32(0.0))

    base = jnp.float32(0.1) + jax.nn.softplus(
        te + jnp.float32(0.1) * dt + jnp.float32(0.01) * jnp.cos(t))

    v0 = jnp.zeros_like(base)
    for k in range(_NUM_TYPES):
        v0 = v0 + (base * jnp.float32(_MU[k]) + jnp.float32(0.05))
    M = v0 * jnp.float32(_OVER)

    sub = jax.lax.broadcasted_iota(jnp.int32, (_E, _TL), 0)
    lane = jax.lax.broadcasted_iota(jnp.int32, (_E, _TL), 1)
    l0 = lt * _TL
    ie = (b * (L * _E) + (l0 + lane) * _E + sub).astype(jnp.uint32)
    u1 = _bits_to_uniform(_threefry_bits(1, ie))
    e = -jnp.log1p(-u1)
    x = e / M
    for s in (1, 2, 4, 8, 16):
        shifted = jnp.concatenate(
            [jnp.zeros((s, _TL), jnp.float32), x[:-s, :]], axis=0)
        x = x + shifted
    exp_j = x

    st = base * jnp.exp(jnp.float32(-0.5) * exp_j)
    intens = jnp.zeros_like(st)
    for k in range(_NUM_TYPES):
        intens = intens + (st * jnp.float32(_MU[k]) + jnp.float32(0.05))

    rows = []
    big = jnp.float32(jnp.inf)
    for k in range(_K):
        iu = (b * (L * _K * _E) + (l0 + lane) * (_K * _E)
              + k * _E + sub).astype(jnp.uint32)
        u = _bits_to_uniform(_threefry_bits(2, iu))
        crit = (u * M) / intens
        cand = jnp.where(crit < jnp.float32(1.0), exp_j, big)
        mval = jnp.min(cand, axis=0, keepdims=True)
        res = jnp.where(mval == big, jnp.float32(0.0),
                        jnp.minimum(mval, jnp.float32(100000.0)))
        rows.append(res)
    out_ref[...] = jnp.concatenate(rows, axis=0)


def kernel(time_seqs, time_delta_seqs, type_seqs, num_sample):
    B, L = time_seqs.shape
    in_spec = pl.BlockSpec((1, 1, _TL), lambda b, lt: (b, 0, lt))
    out = pl.pallas_call(
        functools.partial(_body, L=L),
        grid=(B, L // _TL),
        in_specs=[in_spec, in_spec, in_spec],
        out_specs=pl.BlockSpec((_K, _TL), lambda b, lt: (b, lt)),
        out_shape=jax.ShapeDtypeStruct((B * _K, L), jnp.float32),
        compiler_params=pltpu.CompilerParams(
            dimension_semantics=("parallel", "parallel")),
    )(time_seqs.reshape(B, 1, L), time_delta_seqs.reshape(B, 1, L),
      type_seqs.reshape(B, 1, L))
    res = out.reshape(B, _K, L).transpose(0, 2, 1)
    weights = jnp.ones((B, L, _K), jnp.float32) / num_sample
    return (res, weights)
